# baseline (speedup 1.0000x reference)
import functools

import jax
import jax.numpy as jnp
from jax import lax
from jax.experimental import pallas as pl
from jax.experimental.pallas import tpu as pltpu
from jax.experimental.pallas import tpu_sc as plsc

_NBUF = 4
_CH = 800


@functools.cache
def _build_gather(B2, V2, D2):
    info = plsc.get_sparse_core_info()
    NC, NS = info.num_cores, info.num_subcores
    NW = NC * NS
    assert B2 % NW == 0
    b_per_w = B2 // NW

    CH, NBUF = _CH, _NBUF
    assert b_per_w % (CH * NBUF) == 0
    n_chunks = b_per_w // CH
    n_groups = n_chunks // NBUF

    mesh = plsc.VectorSubcoreMesh(core_axis_name="c", subcore_axis_name="s")

    @functools.partial(
        pl.kernel,
        mesh=mesh,
        out_type=jax.ShapeDtypeStruct((B2, D2), jnp.float32),
        scratch_types=[
            pltpu.VMEM((b_per_w,), jnp.int32),
            pltpu.VMEM((NBUF, CH, D2), jnp.float32),
        ]
        + [pltpu.SemaphoreType.DMA] * (2 * NBUF),
        compiler_params=pltpu.CompilerParams(use_tc_tiling_on_sc=False),
    )
    def gather_kernel(idx_hbm, table_hbm, out_hbm, idx_v, rows_v, *sems):
        sem_g = sems[:NBUF]
        sem_s = sems[NBUF:]
        wid = lax.axis_index("s") * NC + lax.axis_index("c")
        base = wid * b_per_w

        pltpu.sync_copy(idx_hbm.at[pl.ds(base, b_per_w)], idx_v)

        def start_gather(g, b):
            pltpu.async_copy(
                table_hbm.at[idx_v.at[pl.ds(g * CH, CH)]], rows_v.at[b], sem_g[b]
            )

        def wait_gather(b):
            pltpu.make_async_copy(
                table_hbm.at[idx_v.at[pl.ds(0, CH)]], rows_v.at[b], sem_g[b]
            ).wait()

        def start_store(g, b):
            pltpu.async_copy(
                rows_v.at[b], out_hbm.at[pl.ds(base + g * CH, CH)], sem_s[b]
            )

        def wait_store(b):
            pltpu.make_async_copy(
                rows_v.at[b], out_hbm.at[pl.ds(base, CH)], sem_s[b]
            ).wait()

        for b in range(NBUF):
            start_gather(b, b)
            if b >= 1:
                wait_gather(b - 1)
                start_store(b - 1, b - 1)

        def group(s, carry):
            g0 = s * NBUF
            for b in range(NBUF):
                g = g0 + b
                wait_store(b)
                start_gather(g, b)
                pb = (b - 1) % NBUF
                wait_gather(pb)
                start_store(g - 1, pb)
            return carry

        lax.fori_loop(1, n_groups, group, 0)

        last = n_chunks - 1
        lb = last % NBUF
        wait_gather(lb)
        start_store(last, lb)
        for b in range(NBUF):
            wait_store(b)

    return gather_kernel


@functools.cache
def _build_table_transpose(D, V, bk):
    def body(i_ref, o_ref):
        o_ref[...] = i_ref[...].T

    return pl.pallas_call(
        body,
        grid=(pl.cdiv(V, bk),),
        in_specs=[pl.BlockSpec((D, bk), lambda i: (0, i))],
        out_specs=pl.BlockSpec((bk, D), lambda i: (i, 0)),
        out_shape=jax.ShapeDtypeStruct((V, D), jnp.float32),
    )


@functools.cache
def _build_out_transpose(S, B0, D):
    def body(i_ref, o_ref):
        o_ref[0] = i_ref[0].T

    return pl.pallas_call(
        body,
        grid=(S,),
        in_specs=[pl.BlockSpec((1, B0, D), lambda i: (i, 0, 0))],
        out_specs=pl.BlockSpec((1, D, B0), lambda i: (i, 0, 0)),
        out_shape=jax.ShapeDtypeStruct((S, D, B0), jnp.float32),
    )


def kernel(x, W):
    B0, S = x.shape
    V, D = W.shape
    B = B0 * S

    xt = x.T.astype(jnp.int32)
    wt = W.T

    w_rm = _build_table_transpose(D, V, 4096)(wt)

    xf = xt.reshape(B)
    idx2 = (2 * xf[:, None] + jnp.arange(2, dtype=jnp.int32)[None, :]).reshape(2 * B)
    g2 = _build_gather(2 * B, 2 * V, D // 2)(idx2, w_rm.reshape(2 * V, D // 2))

    out_phys = _build_out_transpose(S, B0, D)(g2.reshape(S, B0, D))
    return jnp.transpose(out_phys, (2, 0, 1))


# speedup vs baseline: 1.0019x; 1.0019x over previous
import functools

import jax
import jax.numpy as jnp
from jax import lax
from jax.experimental import pallas as pl
from jax.experimental.pallas import tpu as pltpu
from jax.experimental.pallas import tpu_sc as plsc

_NBUF = 4
_CH = 800


@functools.cache
def _build_gather(B2, V2, D2):
    info = plsc.get_sparse_core_info()
    NC, NS = info.num_cores, info.num_subcores
    NW = NC * NS
    assert B2 % NW == 0
    b_per_w = B2 // NW

    CH, NBUF = _CH, _NBUF
    assert b_per_w % (CH * NBUF) == 0
    n_chunks = b_per_w // CH
    n_groups = n_chunks // NBUF

    mesh = plsc.VectorSubcoreMesh(core_axis_name="c", subcore_axis_name="s")

    @functools.partial(
        pl.kernel,
        mesh=mesh,
        out_type=jax.ShapeDtypeStruct((B2, D2), jnp.float32),
        scratch_types=[
            pltpu.VMEM((b_per_w,), jnp.int32),
            pltpu.VMEM((NBUF, CH, D2), jnp.float32),
        ]
        + [pltpu.SemaphoreType.DMA] * (2 * NBUF),
        compiler_params=pltpu.CompilerParams(use_tc_tiling_on_sc=False),
    )
    def gather_kernel(idx_hbm, table_hbm, out_hbm, idx_v, rows_v, *sems):
        sem_g = sems[:NBUF]
        sem_s = sems[NBUF:]
        wid = lax.axis_index("s") * NC + lax.axis_index("c")
        base = wid * b_per_w

        pltpu.sync_copy(idx_hbm.at[pl.ds(base, b_per_w)], idx_v)

        def start_gather(g, b):
            pltpu.async_copy(
                table_hbm.at[idx_v.at[pl.ds(g * CH, CH)]], rows_v.at[b], sem_g[b]
            )

        def wait_gather(b):
            pltpu.make_async_copy(
                table_hbm.at[idx_v.at[pl.ds(0, CH)]], rows_v.at[b], sem_g[b]
            ).wait()

        def start_store(g, b):
            pltpu.async_copy(
                rows_v.at[b], out_hbm.at[pl.ds(base + g * CH, CH)], sem_s[b]
            )

        def wait_store(b):
            pltpu.make_async_copy(
                rows_v.at[b], out_hbm.at[pl.ds(base, CH)], sem_s[b]
            ).wait()

        for b in range(NBUF):
            start_gather(b, b)
            if b >= 1:
                wait_gather(b - 1)
                start_store(b - 1, b - 1)

        def group(s, carry):
            g0 = s * NBUF
            for b in range(NBUF):
                g = g0 + b
                wait_store(b)
                start_gather(g, b)
                pb = (b - 1) % NBUF
                wait_gather(pb)
                start_store(g - 1, pb)
            return carry

        lax.fori_loop(1, n_groups, group, 0)

        last = n_chunks - 1
        lb = last % NBUF
        wait_gather(lb)
        start_store(last, lb)
        for b in range(NBUF):
            wait_store(b)

    return gather_kernel


@functools.cache
def _build_table_transpose(D, V, bk):
    def body(i_ref, o_ref):
        eye = jnp.eye(D, dtype=jnp.float32)
        o_ref[...] = jax.lax.dot_general(
            i_ref[...], eye, (((0,), (0,)), ((), ())),
            preferred_element_type=jnp.float32,
        )

    return pl.pallas_call(
        body,
        grid=(pl.cdiv(V, bk),),
        in_specs=[pl.BlockSpec((D, bk), lambda i: (0, i))],
        out_specs=pl.BlockSpec((bk, D), lambda i: (i, 0)),
        out_shape=jax.ShapeDtypeStruct((V, D), jnp.float32),
    )


@functools.cache
def _build_out_transpose(S, B0, D):
    def body(i_ref, o_ref):
        eye = jnp.eye(D, dtype=jnp.float32)
        o_ref[0] = jax.lax.dot_general(
            eye, i_ref[0], (((1,), (1,)), ((), ())),
            preferred_element_type=jnp.float32,
        )

    return pl.pallas_call(
        body,
        grid=(S,),
        in_specs=[pl.BlockSpec((1, B0, D), lambda i: (i, 0, 0))],
        out_specs=pl.BlockSpec((1, D, B0), lambda i: (i, 0, 0)),
        out_shape=jax.ShapeDtypeStruct((S, D, B0), jnp.float32),
    )


def kernel(x, W):
    B0, S = x.shape
    V, D = W.shape
    B = B0 * S

    xt = x.T.astype(jnp.int32)
    wt = W.T

    w_rm = _build_table_transpose(D, V, 4096)(wt)

    xf = xt.reshape(B)
    idx2 = (2 * xf[:, None] + jnp.arange(2, dtype=jnp.int32)[None, :]).reshape(2 * B)
    g2 = _build_gather(2 * B, 2 * V, D // 2)(idx2, w_rm.reshape(2 * V, D // 2))

    out_phys = _build_out_transpose(S, B0, D)(g2.reshape(S, B0, D))
    return jnp.transpose(out_phys, (2, 0, 1))


# speedup vs baseline: 2.6479x; 2.6429x over previous
import functools

import jax
import jax.numpy as jnp
from jax import lax
from jax.experimental import pallas as pl
from jax.experimental.pallas import tpu as pltpu
from jax.experimental.pallas import tpu_sc as plsc

_NBUF = 4
_CH = 640
_LANES = 16
_TBK = 8192


@functools.cache
def _build_gather(B, VP, D):
    info = plsc.get_sparse_core_info()
    NC, NS = info.num_cores, info.num_subcores
    NW = NC * NS
    assert B % NW == 0
    b_per_w = B // NW

    CH, NBUF = _CH, _NBUF
    assert b_per_w % (CH * NBUF) == 0
    n_chunks = b_per_w // CH
    n_groups = n_chunks // NBUF

    mesh = plsc.VectorSubcoreMesh(core_axis_name="c", subcore_axis_name="s")

    @functools.partial(
        pl.kernel,
        mesh=mesh,
        out_type=jax.ShapeDtypeStruct((B, D), jnp.float32),
        scratch_types=[
            pltpu.VMEM((b_per_w,), jnp.int32),
            pltpu.VMEM((NBUF, CH, D), jnp.float32),
        ]
        + [pltpu.SemaphoreType.DMA] * (2 * NBUF),
        compiler_params=pltpu.CompilerParams(use_tc_tiling_on_sc=False),
    )
    def gather_kernel(idx_hbm, table_hbm, out_hbm, idx_v, rows_v, *sems):
        sem_g = sems[:NBUF]
        sem_s = sems[NBUF:]
        wid = lax.axis_index("s") * NC + lax.axis_index("c")
        base = wid * b_per_w

        pltpu.sync_copy(idx_hbm.at[pl.ds(base, b_per_w)], idx_v)

        def expand(i, carry):
            v = idx_v[pl.ds(i * _LANES, _LANES)]
            m = ((v >> 13) << 13) + ((v & 2047) << 2) + ((v >> 11) & 3)
            idx_v[pl.ds(i * _LANES, _LANES)] = m
            return carry

        lax.fori_loop(0, b_per_w // _LANES, expand, 0)

        def start_gather(g, b):
            pltpu.async_copy(
                table_hbm.at[idx_v.at[pl.ds(g * CH, CH)]], rows_v.at[b], sem_g[b]
            )

        def wait_gather(b):
            pltpu.make_async_copy(
                table_hbm.at[idx_v.at[pl.ds(0, CH)]], rows_v.at[b], sem_g[b]
            ).wait()

        def start_store(g, b):
            pltpu.async_copy(
                rows_v.at[b], out_hbm.at[pl.ds(base + g * CH, CH)], sem_s[b]
            )

        def wait_store(b):
            pltpu.make_async_copy(
                rows_v.at[b], out_hbm.at[pl.ds(base, CH)], sem_s[b]
            ).wait()

        for b in range(NBUF):
            start_gather(b, b)
            if b >= 1:
                wait_gather(b - 1)
                start_store(b - 1, b - 1)

        def group(s, carry):
            g0 = s * NBUF
            for b in range(NBUF):
                g = g0 + b
                wait_store(b)
                start_gather(g, b)
                pb = (b - 1) % NBUF
                wait_gather(pb)
                start_store(g - 1, pb)
            return carry

        lax.fori_loop(1, n_groups, group, 0)

        last = n_chunks - 1
        lb = last % NBUF
        wait_gather(lb)
        start_store(last, lb)
        for b in range(NBUF):
            wait_store(b)

    return gather_kernel


@functools.cache
def _build_table_transpose(D, V):
    bk = _TBK
    nblk = pl.cdiv(V, bk)
    VP = nblk * bk
    q_per_blk = bk * D // 128
    nr = 128 // D

    def body(i_ref, o_ref):
        eye = jnp.eye(D, dtype=jnp.float32)
        t = jax.lax.dot_general(
            i_ref[...], eye, (((0,), (0,)), ((), ())),
            preferred_element_type=jnp.float32,
        )
        for r in range(nr):
            o_ref[:, r * D:(r + 1) * D] = t[r * q_per_blk:(r + 1) * q_per_blk, :]

    return VP, pl.pallas_call(
        body,
        grid=(nblk,),
        in_specs=[pl.BlockSpec((D, bk), lambda i: (0, i))],
        out_specs=pl.BlockSpec((q_per_blk, 128), lambda i: (i, 0)),
        out_shape=jax.ShapeDtypeStruct((VP * D // 128, 128), jnp.float32),
    )


@functools.cache
def _build_out_transpose(S, B0, D):
    rows_per_s = B0 * D // 256
    nw = 256 // D
    cw = B0 // nw

    def body(i_ref, o_ref):
        eye = jnp.eye(D, dtype=jnp.float32)
        y = i_ref[...]
        for w in range(nw):
            yw = y[:, w * D:(w + 1) * D]
            o_ref[0, :, w * cw:(w + 1) * cw] = jax.lax.dot_general(
                eye, yw, (((1,), (1,)), ((), ())),
                preferred_element_type=jnp.float32,
            )

    return pl.pallas_call(
        body,
        grid=(S,),
        in_specs=[pl.BlockSpec((rows_per_s, 256), lambda i: (i, 0))],
        out_specs=pl.BlockSpec((1, D, B0), lambda i: (i, 0, 0)),
        out_shape=jax.ShapeDtypeStruct((S, D, B0), jnp.float32),
    )


def kernel(x, W):
    B0, S = x.shape
    V, D = W.shape
    B = B0 * S
    nw = 256 // D
    cw = B0 // nw

    xt = x.T.astype(jnp.int32)
    wt = W.T

    VP, table_tr = _build_table_transpose(D, V)
    w128 = table_tr(wt)

    xperm = xt.reshape(S, nw, cw).transpose(0, 2, 1).reshape(B)
    g2 = _build_gather(B, VP, D)(xperm, w128.reshape(VP, D))

    out_phys = _build_out_transpose(S, B0, D)(g2.reshape(S * B0 * D // 256, 256))
    return jnp.transpose(out_phys, (2, 0, 1))


# speedup vs baseline: 3.0632x; 1.1569x over previous
import functools

import jax
import jax.numpy as jnp
from jax import lax
from jax.experimental import pallas as pl
from jax.experimental.pallas import tpu as pltpu
from jax.experimental.pallas import tpu_sc as plsc

_NBUF = 4
_CH = 640
_LANES = 16
_TBK = 32768


@functools.cache
def _build_gather(B, VP, D):
    info = plsc.get_sparse_core_info()
    NC, NS = info.num_cores, info.num_subcores
    NW = NC * NS
    assert B % NW == 0
    b_per_w = B // NW

    CH, NBUF = _CH, _NBUF
    assert b_per_w % (CH * NBUF) == 0
    n_chunks = b_per_w // CH
    n_groups = n_chunks // NBUF

    mesh = plsc.VectorSubcoreMesh(core_axis_name="c", subcore_axis_name="s")

    @functools.partial(
        pl.kernel,
        mesh=mesh,
        out_type=jax.ShapeDtypeStruct((B, D), jnp.float32),
        scratch_types=[
            pltpu.VMEM((b_per_w,), jnp.int32),
            pltpu.VMEM((NBUF, CH, D), jnp.float32),
        ]
        + [pltpu.SemaphoreType.DMA] * (2 * NBUF),
        compiler_params=pltpu.CompilerParams(use_tc_tiling_on_sc=False),
    )
    def gather_kernel(idx_hbm, table_hbm, out_hbm, idx_v, rows_v, *sems):
        sem_g = sems[:NBUF]
        sem_s = sems[NBUF:]
        wid = lax.axis_index("s") * NC + lax.axis_index("c")
        base = wid * b_per_w

        pltpu.sync_copy(idx_hbm.at[pl.ds(base, b_per_w)], idx_v)

        L = _TBK.bit_length() - 1
        Q = _TBK // 4

        def expand(i, carry):
            v = idx_v[pl.ds(i * _LANES, _LANES)]
            m = ((v >> L) << L) + ((v & (Q - 1)) << 2) + ((v >> (L - 2)) & 3)
            idx_v[pl.ds(i * _LANES, _LANES)] = m
            return carry

        lax.fori_loop(0, b_per_w // _LANES, expand, 0)

        def start_gather(g, b):
            pltpu.async_copy(
                table_hbm.at[idx_v.at[pl.ds(g * CH, CH)]], rows_v.at[b], sem_g[b]
            )

        def wait_gather(b):
            pltpu.make_async_copy(
                table_hbm.at[idx_v.at[pl.ds(0, CH)]], rows_v.at[b], sem_g[b]
            ).wait()

        def start_store(g, b):
            pltpu.async_copy(
                rows_v.at[b], out_hbm.at[pl.ds(base + g * CH, CH)], sem_s[b]
            )

        def wait_store(b):
            pltpu.make_async_copy(
                rows_v.at[b], out_hbm.at[pl.ds(base, CH)], sem_s[b]
            ).wait()

        for b in range(NBUF):
            start_gather(b, b)
            if b >= 1:
                wait_gather(b - 1)
                start_store(b - 1, b - 1)

        def group(s, carry):
            g0 = s * NBUF
            for b in range(NBUF):
                g = g0 + b
                wait_store(b)
                start_gather(g, b)
                pb = (b - 1) % NBUF
                wait_gather(pb)
                start_store(g - 1, pb)
            return carry

        lax.fori_loop(1, n_groups, group, 0)

        last = n_chunks - 1
        lb = last % NBUF
        wait_gather(lb)
        start_store(last, lb)
        for b in range(NBUF):
            wait_store(b)

    return gather_kernel


@functools.cache
def _build_table_transpose(D, V):
    bk = _TBK
    nblk = pl.cdiv(V, bk)
    VP = nblk * bk
    q_per_blk = bk * D // 128
    nr = 128 // D

    def body(i_ref, o_ref):
        eye = jnp.eye(D, dtype=jnp.float32)
        t = jax.lax.dot_general(
            i_ref[...], eye, (((0,), (0,)), ((), ())),
            preferred_element_type=jnp.float32,
        )
        for r in range(nr):
            o_ref[:, r * D:(r + 1) * D] = t[r * q_per_blk:(r + 1) * q_per_blk, :]

    return VP, pl.pallas_call(
        body,
        grid=(nblk,),
        in_specs=[pl.BlockSpec((D, bk), lambda i: (0, i))],
        out_specs=pl.BlockSpec((q_per_blk, 128), lambda i: (i, 0)),
        out_shape=jax.ShapeDtypeStruct((VP * D // 128, 128), jnp.float32),
        compiler_params=pltpu.CompilerParams(vmem_limit_bytes=100663296),
    )


@functools.cache
def _build_out_transpose(S, B0, D):
    rows_per_s = B0 * D // 256
    nw = 256 // D
    cw = B0 // nw
    sblk = 4

    def body(i_ref, o_ref):
        eye = jnp.eye(D, dtype=jnp.float32)
        y = i_ref[...]
        for sl in range(sblk):
            ys = y[sl * rows_per_s:(sl + 1) * rows_per_s, :]
            for w in range(nw):
                yw = ys[:, w * D:(w + 1) * D]
                o_ref[sl, :, w * cw:(w + 1) * cw] = jax.lax.dot_general(
                    eye, yw, (((1,), (1,)), ((), ())),
                    preferred_element_type=jnp.float32,
                )

    return pl.pallas_call(
        body,
        grid=(S // sblk,),
        in_specs=[pl.BlockSpec((sblk * rows_per_s, 256), lambda i: (i, 0))],
        out_specs=pl.BlockSpec((sblk, D, B0), lambda i: (i, 0, 0)),
        out_shape=jax.ShapeDtypeStruct((S, D, B0), jnp.float32),
        compiler_params=pltpu.CompilerParams(vmem_limit_bytes=100663296),
    )


def kernel(x, W):
    B0, S = x.shape
    V, D = W.shape
    B = B0 * S
    nw = 256 // D
    cw = B0 // nw

    xt = x.T.astype(jnp.int32)
    wt = W.T

    VP, table_tr = _build_table_transpose(D, V)
    w128 = table_tr(wt)

    xperm = xt.reshape(S, nw, cw).transpose(0, 2, 1).reshape(B)
    g2 = _build_gather(B, VP, D)(xperm, w128.reshape(VP, D))

    out_phys = _build_out_transpose(S, B0, D)(g2.reshape(S * B0 * D // 256, 256))
    return jnp.transpose(out_phys, (2, 0, 1))


# speedup vs baseline: 3.7111x; 1.2115x over previous
import functools

import jax
import jax.numpy as jnp
from jax import lax
from jax.experimental import pallas as pl
from jax.experimental.pallas import tpu as pltpu
from jax.experimental.pallas import tpu_sc as plsc

_NBUF = 4
_CH = 640
_LANES = 16
_TBK = 32768


@functools.cache
def _build_gather(B, VP, D):
    info = plsc.get_sparse_core_info()
    NC, NS = info.num_cores, info.num_subcores
    NW = NC * NS
    assert B % NW == 0
    b_per_w = B // NW

    CH, NBUF = _CH, _NBUF
    assert b_per_w % (CH * NBUF) == 0
    n_chunks = b_per_w // CH
    n_groups = n_chunks // NBUF

    mesh = plsc.VectorSubcoreMesh(core_axis_name="c", subcore_axis_name="s")

    @functools.partial(
        pl.kernel,
        mesh=mesh,
        out_type=jax.ShapeDtypeStruct((B, D), jnp.float32),
        scratch_types=[
            pltpu.VMEM((b_per_w,), jnp.int32),
            pltpu.VMEM((NBUF, CH, D), jnp.float32),
        ]
        + [pltpu.SemaphoreType.DMA] * (2 * NBUF),
        compiler_params=pltpu.CompilerParams(use_tc_tiling_on_sc=False),
    )
    def gather_kernel(idx_hbm, table_hbm, out_hbm, idx_v, rows_v, *sems):
        sem_g = sems[:NBUF]
        sem_s = sems[NBUF:]
        wid = lax.axis_index("s") * NC + lax.axis_index("c")
        base = wid * b_per_w

        pltpu.sync_copy(idx_hbm.at[pl.ds(base, b_per_w)], idx_v)

        L = _TBK.bit_length() - 1
        Q = _TBK // 4

        def expand(i, carry):
            v = idx_v[pl.ds(i * _LANES, _LANES)]
            m = ((v >> L) << L) + ((v & (Q - 1)) << 2) + ((v >> (L - 2)) & 3)
            idx_v[pl.ds(i * _LANES, _LANES)] = m
            return carry

        lax.fori_loop(0, b_per_w // _LANES, expand, 0)

        def start_gather(g, b):
            pltpu.async_copy(
                table_hbm.at[idx_v.at[pl.ds(g * CH, CH)]], rows_v.at[b], sem_g[b]
            )

        def wait_gather(b):
            pltpu.make_async_copy(
                table_hbm.at[idx_v.at[pl.ds(0, CH)]], rows_v.at[b], sem_g[b]
            ).wait()

        def start_store(g, b):
            pltpu.async_copy(
                rows_v.at[b], out_hbm.at[pl.ds(base + g * CH, CH)], sem_s[b]
            )

        def wait_store(b):
            pltpu.make_async_copy(
                rows_v.at[b], out_hbm.at[pl.ds(base, CH)], sem_s[b]
            ).wait()

        for b in range(NBUF):
            start_gather(b, b)
            if b >= 1:
                wait_gather(b - 1)
                start_store(b - 1, b - 1)

        def group(s, carry):
            g0 = s * NBUF
            for b in range(NBUF):
                g = g0 + b
                wait_store(b)
                start_gather(g, b)
                pb = (b - 1) % NBUF
                wait_gather(pb)
                start_store(g - 1, pb)
            return carry

        lax.fori_loop(1, n_groups, group, 0)

        last = n_chunks - 1
        lb = last % NBUF
        wait_gather(lb)
        start_store(last, lb)
        for b in range(NBUF):
            wait_store(b)

    return gather_kernel


@functools.cache
def _build_table_transpose(D, V):
    bk = _TBK
    nblk = pl.cdiv(V, bk)
    VP = nblk * bk
    q_per_blk = bk * D // 128
    nr = 128 // D

    def body(i_ref, o_ref):
        eye = jnp.eye(D, dtype=jnp.float32)
        t = jax.lax.dot_general(
            i_ref[...], eye, (((0,), (0,)), ((), ())),
            preferred_element_type=jnp.float32,
        )
        for r in range(nr):
            o_ref[:, r * D:(r + 1) * D] = t[r * q_per_blk:(r + 1) * q_per_blk, :]

    return VP, pl.pallas_call(
        body,
        grid=(nblk,),
        in_specs=[pl.BlockSpec((D, bk), lambda i: (0, i))],
        out_specs=pl.BlockSpec((q_per_blk, 128), lambda i: (i, 0)),
        out_shape=jax.ShapeDtypeStruct((VP * D // 128, 128), jnp.float32),
        compiler_params=pltpu.CompilerParams(vmem_limit_bytes=100663296),
    )


@functools.cache
def _build_out_transpose(S, B0, D):
    rows_per_s = B0 * D // 256
    nw = 256 // D
    cw = B0 // nw
    sblk = 4

    def body(i_ref, o_ref):
        eye = jnp.eye(D, dtype=jnp.float32)
        y = i_ref[...].reshape(sblk * rows_per_s, 256)
        for sl in range(sblk):
            ys = y[sl * rows_per_s:(sl + 1) * rows_per_s, :]
            for w in range(nw):
                yw = ys[:, w * D:(w + 1) * D]
                o_ref[sl, :, w * cw:(w + 1) * cw] = jax.lax.dot_general(
                    eye, yw, (((1,), (1,)), ((), ())),
                    preferred_element_type=jnp.float32,
                )

    return pl.pallas_call(
        body,
        grid=(S // sblk,),
        in_specs=[pl.BlockSpec((sblk * rows_per_s * 256,), lambda i: (i,))],
        out_specs=pl.BlockSpec((sblk, D, B0), lambda i: (i, 0, 0)),
        out_shape=jax.ShapeDtypeStruct((S, D, B0), jnp.float32),
        compiler_params=pltpu.CompilerParams(vmem_limit_bytes=100663296),
    )


def kernel(x, W):
    B0, S = x.shape
    V, D = W.shape
    B = B0 * S
    nw = 256 // D
    cw = B0 // nw

    xt = x.T.astype(jnp.int32)
    wt = W.T

    VP, table_tr = _build_table_transpose(D, V)
    w128 = table_tr(wt)

    xperm = xt.reshape(S, nw, cw).transpose(0, 2, 1).reshape(B)
    g2 = _build_gather(B, VP, D)(xperm, w128.reshape(VP, D))

    out_phys = _build_out_transpose(S, B0, D)(g2.reshape(B * D))
    return jnp.transpose(out_phys, (2, 0, 1))


# speedup vs baseline: 3.7162x; 1.0014x over previous
import functools

import jax
import jax.numpy as jnp
from jax import lax
from jax.experimental import pallas as pl
from jax.experimental.pallas import tpu as pltpu
from jax.experimental.pallas import tpu_sc as plsc

_NBUF = 4
_CH = 640
_LANES = 16
_TBK = 32768


@functools.cache
def _build_gather(B, VP, D):
    info = plsc.get_sparse_core_info()
    NC, NS = info.num_cores, info.num_subcores
    NW = NC * NS
    assert B % NW == 0
    b_per_w = B // NW

    CH, NBUF = _CH, _NBUF
    assert b_per_w % (CH * NBUF) == 0
    n_chunks = b_per_w // CH
    n_groups = n_chunks // NBUF

    mesh = plsc.VectorSubcoreMesh(core_axis_name="c", subcore_axis_name="s")

    @functools.partial(
        pl.kernel,
        mesh=mesh,
        out_type=jax.ShapeDtypeStruct((B, D), jnp.float32),
        scratch_types=[
            pltpu.VMEM((b_per_w,), jnp.int32),
            pltpu.VMEM((NBUF, CH, D), jnp.float32),
        ]
        + [pltpu.SemaphoreType.DMA] * (2 * NBUF),
        compiler_params=pltpu.CompilerParams(use_tc_tiling_on_sc=False),
    )
    def gather_kernel(idx_hbm, table_hbm, out_hbm, idx_v, rows_v, *sems):
        sem_g = sems[:NBUF]
        sem_s = sems[NBUF:]
        wid = lax.axis_index("s") * NC + lax.axis_index("c")
        base = wid * b_per_w

        pltpu.sync_copy(idx_hbm.at[pl.ds(base, b_per_w)], idx_v)

        L = _TBK.bit_length() - 1
        Q = _TBK // 4

        def expand(i, carry):
            v = idx_v[pl.ds(i * _LANES, _LANES)]
            m = ((v >> L) << L) + ((v & (Q - 1)) << 2) + ((v >> (L - 2)) & 3)
            idx_v[pl.ds(i * _LANES, _LANES)] = m
            return carry

        lax.fori_loop(0, b_per_w // _LANES, expand, 0)

        def start_gather(g, b):
            pltpu.async_copy(
                table_hbm.at[idx_v.at[pl.ds(g * CH, CH)]], rows_v.at[b], sem_g[b]
            )

        def wait_gather(b):
            pltpu.make_async_copy(
                table_hbm.at[idx_v.at[pl.ds(0, CH)]], rows_v.at[b], sem_g[b]
            ).wait()

        def start_store(g, b):
            pltpu.async_copy(
                rows_v.at[b], out_hbm.at[pl.ds(base + g * CH, CH)], sem_s[b]
            )

        def wait_store(b):
            pltpu.make_async_copy(
                rows_v.at[b], out_hbm.at[pl.ds(base, CH)], sem_s[b]
            ).wait()

        for b in range(NBUF):
            start_gather(b, b)
            if b >= 1:
                wait_gather(b - 1)
                start_store(b - 1, b - 1)

        def group(s, carry):
            g0 = s * NBUF
            for b in range(NBUF):
                g = g0 + b
                wait_store(b)
                start_gather(g, b)
                pb = (b - 1) % NBUF
                wait_gather(pb)
                start_store(g - 1, pb)
            return carry

        lax.fori_loop(1, n_groups, group, 0)

        last = n_chunks - 1
        lb = last % NBUF
        wait_gather(lb)
        start_store(last, lb)
        for b in range(NBUF):
            wait_store(b)

    return gather_kernel


@functools.cache
def _build_table_transpose(D, V):
    bk = _TBK
    nblk = pl.cdiv(V, bk)
    VP = nblk * bk
    q_per_blk = bk * D // 128
    nr = 128 // D

    def body(i_ref, o_ref):
        eye = jnp.eye(D, dtype=jnp.float32)
        for r in range(nr):
            o_ref[:, r * D:(r + 1) * D] = jax.lax.dot_general(
                i_ref[:, r * q_per_blk:(r + 1) * q_per_blk], eye,
                (((0,), (0,)), ((), ())),
                preferred_element_type=jnp.float32,
            )

    return VP, pl.pallas_call(
        body,
        grid=(nblk,),
        in_specs=[pl.BlockSpec((D, bk), lambda i: (0, i))],
        out_specs=pl.BlockSpec((q_per_blk, 128), lambda i: (i, 0)),
        out_shape=jax.ShapeDtypeStruct((VP * D // 128, 128), jnp.float32),
        compiler_params=pltpu.CompilerParams(vmem_limit_bytes=100663296),
    )


@functools.cache
def _build_out_transpose(S, B0, D):
    rows_per_s = B0 * D // 256
    nw = 256 // D
    cw = B0 // nw
    sblk = 4

    def body(i_ref, o_ref):
        eye = jnp.eye(D, dtype=jnp.float32)
        y = i_ref[...].reshape(sblk * rows_per_s, 256)
        for sl in range(sblk):
            ys = y[sl * rows_per_s:(sl + 1) * rows_per_s, :]
            for w in range(nw):
                yw = ys[:, w * D:(w + 1) * D]
                o_ref[sl, :, w * cw:(w + 1) * cw] = jax.lax.dot_general(
                    eye, yw, (((1,), (1,)), ((), ())),
                    preferred_element_type=jnp.float32,
                )

    return pl.pallas_call(
        body,
        grid=(S // sblk,),
        in_specs=[pl.BlockSpec((sblk * rows_per_s * 256,), lambda i: (i,))],
        out_specs=pl.BlockSpec((sblk, D, B0), lambda i: (i, 0, 0)),
        out_shape=jax.ShapeDtypeStruct((S, D, B0), jnp.float32),
        compiler_params=pltpu.CompilerParams(vmem_limit_bytes=100663296),
    )


def kernel(x, W):
    B0, S = x.shape
    V, D = W.shape
    B = B0 * S
    nw = 256 // D
    cw = B0 // nw

    xt = x.T.astype(jnp.int32)
    wt = W.T

    VP, table_tr = _build_table_transpose(D, V)
    w128 = table_tr(wt)

    xperm = xt.reshape(S, nw, cw).transpose(0, 2, 1).reshape(B)
    g2 = _build_gather(B, VP, D)(xperm, w128.reshape(VP, D))

    out_phys = _build_out_transpose(S, B0, D)(g2.reshape(B * D))
    return jnp.transpose(out_phys, (2, 0, 1))
